# Initial kernel scaffold; baseline (speedup 1.0000x reference)
import functools

import jax
import jax.numpy as jnp
from jax import lax
from jax.experimental import pallas as pl
from jax.experimental.pallas import tpu as pltpu
from jax.experimental.pallas import tpu_sc as plsc

NC = 2
NS = 16
NW = NC * NS
LANES = 16


def _make_sc_gather_dot(B, D, NEG):
    per_w = B // NW
    BLK = 128
    nblk = per_w // BLK
    grp = BLK // LANES

    mesh = plsc.VectorSubcoreMesh(core_axis_name="c", subcore_axis_name="s")

    @functools.partial(
        pl.kernel,
        out_type=[
            jax.ShapeDtypeStruct((B,), jnp.float32),
            jax.ShapeDtypeStruct((NEG, B), jnp.float32),
        ],
        mesh=mesh,
        scratch_types=[
            pltpu.VMEM((BLK,), jnp.int32),
            pltpu.VMEM((BLK,), jnp.int32),
            pltpu.VMEM((BLK * NEG,), jnp.int32),
            pltpu.VMEM((BLK, D), jnp.float32),
            pltpu.VMEM((BLK, D), jnp.float32),
            pltpu.VMEM((BLK * NEG, D), jnp.float32),
            pltpu.VMEM((BLK,), jnp.float32),
            pltpu.VMEM((NEG, BLK), jnp.float32),
            pltpu.SemaphoreType.DMA,
        ],
    )
    def sc_fn(pos_u_h, pos_v_h, negf_h, uw_h, vw_h, score_h, negsc_h,
              idx_u, idx_v, idx_n, rows_u, rows_v, rows_n, score_v, negsc_v,
              sem):
        wid = lax.axis_index("s") * NC + lax.axis_index("c")
        base_w = wid * per_w
        iota = lax.iota(jnp.int32, LANES)

        for blk in range(nblk):
            base = base_w + blk * BLK
            pltpu.sync_copy(pos_u_h.at[pl.ds(base, BLK)], idx_u)
            pltpu.sync_copy(pos_v_h.at[pl.ds(base, BLK)], idx_v)
            pltpu.sync_copy(negf_h.at[pl.ds(base * NEG, BLK * NEG)], idx_n)
            cu = pltpu.async_copy(uw_h.at[idx_u], rows_u, sem)
            cv = pltpu.async_copy(vw_h.at[idx_v], rows_v, sem)
            cns = [
                pltpu.async_copy(
                    vw_h.at[idx_n.at[pl.ds(k * BLK, BLK)]],
                    rows_n.at[pl.ds(k * BLK, BLK)], sem)
                for k in range(NEG)
            ]
            cu.wait()
            cv.wait()
            for c in cns:
                c.wait()

            for g in range(grp):
                row = g * LANES + iota

                def dstep(d, carry, row=row):
                    dv = jnp.full((LANES,), d, jnp.int32)
                    uv = plsc.load_gather(rows_u, [row, dv])
                    vv = plsc.load_gather(rows_v, [row, dv])
                    acc = carry[0] + uv * vv
                    nbase = row * NEG
                    outs = [acc]
                    for n in range(NEG):
                        nv = plsc.load_gather(rows_n, [nbase + n, dv])
                        outs.append(carry[n + 1] + nv * uv)
                    return tuple(outs)

                init = tuple(jnp.zeros((LANES,), jnp.float32)
                             for _ in range(NEG + 1))
                res = lax.fori_loop(0, D, dstep, init)
                score_v[pl.ds(g * LANES, LANES)] = res[0]
                for n in range(NEG):
                    negsc_v[n, pl.ds(g * LANES, LANES)] = res[n + 1]

            pltpu.sync_copy(score_v, score_h.at[pl.ds(base, BLK)])
            for n in range(NEG):
                pltpu.sync_copy(negsc_v.at[n], negsc_h.at[n, pl.ds(base, BLK)])

    return sc_fn


def _softplus(x):
    return jnp.maximum(x, 0.0) + jnp.log1p(jnp.exp(-jnp.abs(x)))


def _tc_loss_body(score_ref, negsc_ref, out_ref):
    s = jnp.clip(score_ref[...], -10.0, 10.0)
    l1 = jnp.sum(_softplus(-s))
    t = jnp.clip(negsc_ref[...], -10.0, 10.0)
    l2 = jnp.sum(_softplus(t))
    n = score_ref.shape[0] * score_ref.shape[1]
    out_ref[0, 0] = (l1 + l2) / n


def kernel(pos_u, pos_v, neg_v, u_weight, v_weight):
    B = pos_u.shape[0]
    NEG = neg_v.shape[1]
    D = u_weight.shape[1]

    sc_fn = _make_sc_gather_dot(B, D, NEG)
    score, negsc = sc_fn(pos_u, pos_v, neg_v.reshape(-1), u_weight, v_weight)

    score2d = score.reshape(B // 128, 128)
    neg2d = negsc.reshape(NEG * B // 128, 128)
    out = pl.pallas_call(
        _tc_loss_body,
        out_shape=jax.ShapeDtypeStruct((1, 1), jnp.float32),
        out_specs=pl.BlockSpec(memory_space=pltpu.SMEM),
    )(score2d, neg2d)
    return out[0, 0]


# speedup vs baseline: 2.2736x; 2.2736x over previous
import functools

import jax
import jax.numpy as jnp
from jax import lax
from jax.experimental import pallas as pl
from jax.experimental.pallas import tpu as pltpu
from jax.experimental.pallas import tpu_sc as plsc

NC = 2
NS = 16
NW = NC * NS
LANES = 16


def _make_sc_gather_dot(B, D, NEG):
    per_w = B // NW
    BLK = 64
    nblk = per_w // BLK
    grp = BLK // LANES

    mesh = plsc.VectorSubcoreMesh(core_axis_name="c", subcore_axis_name="s")

    @functools.partial(
        pl.kernel,
        out_type=[
            jax.ShapeDtypeStruct((B,), jnp.float32),
            jax.ShapeDtypeStruct((NEG * B,), jnp.float32),
        ],
        mesh=mesh,
        compiler_params=pltpu.CompilerParams(needs_layout_passes=False),
        scratch_types=[
            pltpu.VMEM((BLK,), jnp.int32),
            pltpu.VMEM((BLK,), jnp.int32),
            pltpu.VMEM((BLK * NEG,), jnp.int32),
            pltpu.VMEM((BLK, D), jnp.float32),
            pltpu.VMEM((BLK, D), jnp.float32),
            pltpu.VMEM((BLK * NEG, D), jnp.float32),
            pltpu.VMEM((BLK,), jnp.float32),
            pltpu.VMEM((NEG, BLK), jnp.float32),
            pltpu.SemaphoreType.DMA,
        ],
    )
    def sc_fn(pos_u_h, pos_v_h, negf_h, uw_h, vw_h, score_h, negsc_h,
              idxu, idxv, idxn, rows_u, rows_v, rows_n,
              score_v, negsc_v, sem):
        wid = lax.axis_index("s") * NC + lax.axis_index("c")
        iota = lax.iota(jnp.int32, LANES)

        def block(blk, carry):
            base = pl.multiple_of(wid * per_w + blk * BLK, BLK)
            base5 = pl.multiple_of(base * NEG, BLK * NEG)
            pltpu.sync_copy(pos_u_h.at[pl.ds(base, BLK)], idxu)
            pltpu.sync_copy(pos_v_h.at[pl.ds(base, BLK)], idxv)
            pltpu.sync_copy(negf_h.at[pl.ds(base5, BLK * NEG)], idxn)

            def fire(src_idx, table, dst, ngroups):
                def body(g, c):
                    off = pl.multiple_of(g * LANES, LANES)
                    vec = src_idx[pl.ds(off, LANES)]
                    for i in range(LANES):
                        r = vec[i]
                        pltpu.async_copy(table.at[pl.ds(r, 1)],
                                         dst.at[pl.ds(off + i, 1)], sem)
                    return c
                lax.fori_loop(0, ngroups, body, 0)

            fire(idxu, uw_h, rows_u, BLK // LANES)
            fire(idxv, vw_h, rows_v, BLK // LANES)
            fire(idxn, vw_h, rows_n, BLK * NEG // LANES)
            pltpu.make_async_copy(uw_h.at[pl.ds(0, BLK)], rows_u, sem).wait()
            pltpu.make_async_copy(uw_h.at[pl.ds(0, BLK)], rows_v, sem).wait()
            pltpu.make_async_copy(uw_h.at[pl.ds(0, BLK * NEG)], rows_n,
                                  sem).wait()

            for g in range(grp):
                s = pl.ds(g * LANES, LANES)
                bvec = g * LANES + iota

                def dstep(d, car, bvec=bvec):
                    dv = jnp.full((LANES,), d, jnp.int32)
                    uv = plsc.load_gather(rows_u, [bvec, dv])
                    vv = plsc.load_gather(rows_v, [bvec, dv])
                    outs = [car[0] + uv * vv]
                    for j in range(NEG):
                        nv = plsc.load_gather(rows_n, [bvec * NEG + j, dv])
                        outs.append(car[j + 1] + nv * uv)
                    return tuple(outs)

                init = tuple(jnp.zeros((LANES,), jnp.float32)
                             for _ in range(NEG + 1))
                res = lax.fori_loop(0, D, dstep, init)
                score_v[s] = res[0]
                for j in range(NEG):
                    negsc_v[j, s] = res[j + 1]

            pltpu.sync_copy(score_v, score_h.at[pl.ds(base, BLK)])
            for j in range(NEG):
                off = pl.multiple_of(j * B + base, BLK)
                pltpu.sync_copy(negsc_v.at[j], negsc_h.at[pl.ds(off, BLK)])
            return carry

        lax.fori_loop(0, nblk, block, 0)

    return sc_fn


def _softplus(x):
    return jnp.maximum(x, 0.0) + jnp.log1p(jnp.exp(-jnp.abs(x)))


def _tc_loss_body(score_ref, negsc_ref, out_ref):
    s = jnp.clip(score_ref[...], -10.0, 10.0)
    l1 = jnp.sum(_softplus(-s))
    t = jnp.clip(negsc_ref[...], -10.0, 10.0)
    l2 = jnp.sum(_softplus(t))
    n = score_ref.shape[0] * score_ref.shape[1]
    out_ref[0, 0] = (l1 + l2) / n


def kernel(pos_u, pos_v, neg_v, u_weight, v_weight):
    B = pos_u.shape[0]
    NEG = neg_v.shape[1]
    D = u_weight.shape[1]

    sc_fn = _make_sc_gather_dot(B, D, NEG)
    score, negsc = sc_fn(pos_u, pos_v, neg_v.reshape(-1), u_weight, v_weight)

    score2d = score.reshape(B // 128, 128)
    neg2d = negsc.reshape(NEG * B // 128, 128)
    out = pl.pallas_call(
        _tc_loss_body,
        out_shape=jax.ShapeDtypeStruct((1, 1), jnp.float32),
        out_specs=pl.BlockSpec(memory_space=pltpu.SMEM),
    )(score2d, neg2d)
    return out[0, 0]
